# baseline (speedup 1.0000x reference)
import functools

import jax
import jax.numpy as jnp
from jax import lax
from jax.experimental import pallas as pl
from jax.experimental.pallas import tpu as pltpu
from jax.experimental.pallas import tpu_sc as plsc


@functools.cache
def _build_gather(B, D, nc, ns):
    nw = nc * ns
    b_per_w = B // nw
    chunk = min(128, b_per_w)
    n_chunks = b_per_w // chunk

    mesh = plsc.VectorSubcoreMesh(core_axis_name="c", subcore_axis_name="s")

    @functools.partial(
        pl.kernel,
        mesh=mesh,
        compiler_params=pltpu.CompilerParams(use_tc_tiling_on_sc=False),
        out_type=jax.ShapeDtypeStruct((B, D), jnp.float32),
        scratch_types=[
            pltpu.VMEM((chunk,), jnp.int32),
            pltpu.VMEM((chunk, D), jnp.float32),
            pltpu.SemaphoreType.DMA,
        ],
    )
    def gather(idx_hbm, table_hbm, out_hbm, idx_v, rows_v, sem):
        wid = lax.axis_index("s") * nc + lax.axis_index("c")
        base = wid * b_per_w
        for j in range(n_chunks):
            off = base + j * chunk
            pltpu.sync_copy(idx_hbm.at[pl.ds(off, chunk)], idx_v)
            pltpu.async_copy(table_hbm.at[idx_v], rows_v, sem).wait()
            pltpu.sync_copy(rows_v, out_hbm.at[pl.ds(off, chunk)])

    return gather


def kernel(entity_ids, entity_embedding):
    (B,) = entity_ids.shape
    _, D = entity_embedding.shape
    info = plsc.get_sparse_core_info()
    gather = _build_gather(B, D, info.num_cores, info.num_subcores)
    return gather(entity_ids.astype(jnp.int32), entity_embedding)


# speedup vs baseline: 1.0078x; 1.0078x over previous
import functools

import jax
import jax.numpy as jnp
from jax import lax
from jax.experimental import pallas as pl
from jax.experimental.pallas import tpu as pltpu
from jax.experimental.pallas import tpu_sc as plsc


@functools.cache
def _build_gather(B, D, nc, ns):
    nw = nc * ns
    b_per_w = B // nw

    mesh = plsc.VectorSubcoreMesh(core_axis_name="c", subcore_axis_name="s")

    @functools.partial(
        pl.kernel,
        mesh=mesh,
        compiler_params=pltpu.CompilerParams(use_tc_tiling_on_sc=False),
        out_type=jax.ShapeDtypeStruct((B, D), jnp.float32),
        scratch_types=[
            pltpu.VMEM((b_per_w,), jnp.int32),
            pltpu.VMEM((b_per_w, D), jnp.float32),
            pltpu.SemaphoreType.DMA,
        ],
    )
    def gather(idx_hbm, table_hbm, out_hbm, idx_v, rows_v, sem):
        wid = lax.axis_index("s") * nc + lax.axis_index("c")
        base = wid * b_per_w
        pltpu.sync_copy(idx_hbm.at[pl.ds(base, b_per_w)], idx_v)
        pltpu.async_copy(table_hbm.at[idx_v], rows_v, sem).wait()
        pltpu.sync_copy(rows_v, out_hbm.at[pl.ds(base, b_per_w)])

    return gather


def kernel(entity_ids, entity_embedding):
    (B,) = entity_ids.shape
    _, D = entity_embedding.shape
    info = plsc.get_sparse_core_info()
    gather = _build_gather(B, D, info.num_cores, info.num_subcores)
    return gather(entity_ids.astype(jnp.int32), entity_embedding)


# speedup vs baseline: 2.2311x; 2.2139x over previous
import functools

import jax
import jax.numpy as jnp
from jax import lax
from jax.experimental import pallas as pl
from jax.experimental.pallas import tpu as pltpu
from jax.experimental.pallas import tpu_sc as plsc


@functools.cache
def _build_gather(B, V, D, nc, ns):
    nw = nc * ns
    b_per_w = B // nw
    ch = 32
    n_chunks = b_per_w // ch

    mesh = plsc.VectorSubcoreMesh(core_axis_name="c", subcore_axis_name="s")

    @functools.partial(
        pl.kernel,
        mesh=mesh,
        out_type=jax.ShapeDtypeStruct((B, D), jnp.float32),
        scratch_types=[
            pltpu.VMEM((b_per_w,), jnp.int32),
            pltpu.VMEM((b_per_w,), jnp.int32),
            pltpu.VMEM((ch, 8, D), jnp.float32),
            pltpu.VMEM((ch, D), jnp.float32),
            pltpu.SemaphoreType.DMA,
        ],
    )
    def gather(idx_hbm, table_hbm, out_hbm, blk_v, row_v,
               rows0, outb0, sem0):
        wid = lax.axis_index("s") * nc + lax.axis_index("c")
        base = wid * b_per_w
        pltpu.sync_copy(idx_hbm.at[pl.ds(base, b_per_w)], blk_v)
        for t in range(b_per_w // 16):
            v = blk_v[pl.ds(t * 16, 16)]
            row_v[pl.ds(t * 16, 16)] = lax.bitwise_and(v, 7)
            blk_v[pl.ds(t * 16, 16)] = lax.shift_right_logical(v, 3)

        def chunk_body(j, carry):
            for g in range(ch // 16):
                v16 = blk_v[pl.ds(j * ch + g * 16, 16)]
                for lane in range(16):
                    n = g * 16 + lane
                    pltpu.async_copy(
                        table_hbm.at[v16[lane]], rows0.at[n], sem0)
            pltpu.make_async_copy(
                table_hbm.at[pl.ds(0, ch)], rows0, sem0).wait()
            for g in range(ch // 16):
                p16 = row_v[pl.ds(j * ch + g * 16, 16)]
                for lane in range(16):
                    n = g * 16 + lane
                    p = p16[lane]
                    for c in range(D // 16):
                        outb0[n, pl.ds(c * 16, 16)] = (
                            rows0[n, p, pl.ds(c * 16, 16)])
            pltpu.sync_copy(outb0, out_hbm.at[pl.ds(base + j * ch, ch)])
            return carry

        lax.fori_loop(0, n_chunks, chunk_body, 0)

    return gather


def kernel(entity_ids, entity_embedding):
    (B,) = entity_ids.shape
    V, D = entity_embedding.shape
    info = plsc.get_sparse_core_info()
    gather = _build_gather(B, V, D, info.num_cores, info.num_subcores)
    table3 = entity_embedding.reshape(V // 8, 8, D)
    return gather(entity_ids.astype(jnp.int32), table3)
